# baseline (speedup 1.0000x reference)
import functools

import numpy as np
import jax
import jax.numpy as jnp
from jax.experimental import pallas as pl
from jax.experimental.pallas import tpu as pltpu
from jax.experimental.pallas import tpu_sc as plsc

_STRUC_VOCAB = 516
_SEQ_VOCAB = 33
_STRUC_MASK = 2
_SEQ_MASK = 32
_EPS = 1e-6
_TINY = float(np.finfo(np.float32).tiny)
_BIG = 1 << 30

_KEY_STRUC = (1832780943, 270669613)
_KEY_SEQ = (64467757, 2916123636)

_ROT_A = (13, 15, 26, 6)
_ROT_B = (17, 29, 16, 24)


def _rotl(x, d):
    return (x << jnp.uint32(d)) | (x >> jnp.uint32(32 - d))


def _mbits_core(k0, k1, k2, x1):
    schedule = ((_ROT_A, k1, k2, 1), (_ROT_B, k2, k0, 2),
                (_ROT_A, k0, k1, 3), (_ROT_B, k1, k2, 4),
                (_ROT_A, k2, k0, 5))
    x0 = k0 + x1
    x1 = x0 ^ _rotl(x1, _ROT_A[0])
    first = True
    for rots, ka, kb, c in schedule:
        for r in rots:
            if first:
                first = False
                continue
            x0 = x0 + x1
            x1 = _rotl(x1, r)
            x1 = x0 ^ x1
        x0 = x0 + ka
        x1 = x1 + kb + jnp.uint32(c)
    return ((x0 ^ x1) >> jnp.uint32(9)).astype(jnp.int32)


def _key_consts(key):
    k0, k1 = key
    return jnp.uint32(k0), jnp.uint32(k1), jnp.uint32(k0 ^ k1 ^ 0x1BD11BDA)


def _gumbel_from_mbits(mbits):
    tiny = jnp.float32(_TINY)
    scale = jnp.float32(1.0) - tiny
    fb = jax.lax.bitcast_convert_type(
        mbits | jnp.int32(0x3F800000), jnp.float32) - jnp.float32(1.0)
    u = jnp.maximum(tiny, fb * scale + tiny)
    return -jnp.log(-jnp.log(u))


def _scan_others(nb1, x0, *, n_vocab, n_pad, mask_tok, width, key, chunk):
    k0, k1, k2 = _key_consts(key)
    s_idx = jax.lax.broadcasted_iota(jnp.int32, (chunk, width), 0)
    s_u = jax.lax.broadcasted_iota(jnp.uint32, (chunk, width), 0)
    acc = jnp.full((chunk, width), -1, jnp.int32)
    first_c = jnp.zeros((chunk, width), jnp.int32)
    for c in range(n_pad // chunk):
        n0 = c * chunk
        x1 = nb1 + (s_u + jnp.uint32(n0))
        mb = _mbits_core(k0, k1, k2, x1)
        n_ids = s_idx + n0
        excl = n_ids == x0
        if n0 <= mask_tok < n0 + chunk:
            excl |= n_ids == mask_tok
        if n0 + chunk > n_vocab:
            excl |= n_ids >= n_vocab
        masked = jnp.where(excl, jnp.int32(-1), mb)
        upd = masked > acc
        acc = jnp.where(upd, masked, acc)
        first_c = jnp.where(upd, jnp.int32(c), first_c)
    best_m = jnp.max(acc, axis=0, keepdims=True)
    n_cand = first_c * chunk + s_idx
    elig = acc == best_m
    n_oth = jnp.min(jnp.where(elig, n_cand, jnp.int32(_BIG)),
                    axis=0, keepdims=True)
    return best_m, n_oth


def _pick(a, x0, g_x0, g_mask, g_oth, n_oth, mask_tok):
    eps = jnp.float32(_EPS)
    one = jnp.float32(1.0)
    x0_is_mask = x0 == mask_tok
    p_mask = jnp.where(x0_is_mask, a + (one - a), one - a)
    c_x0 = jnp.where(x0_is_mask, jnp.float32(-jnp.inf),
                     g_x0 + jnp.log(a + eps))
    c_mask = g_mask + jnp.log(p_mask + eps)
    c_oth = g_oth + jnp.log(a * jnp.float32(0.0) + eps)

    best_v, best_i = c_x0, x0
    mask_i = jnp.full_like(best_i, mask_tok)
    take = (c_mask > best_v) | ((c_mask == best_v) & (mask_i < best_i))
    best_v = jnp.where(take, c_mask, best_v)
    best_i = jnp.where(take, mask_i, best_i)
    take = (c_oth > best_v) | ((c_oth == best_v) & (n_oth < best_i))
    return jnp.where(take, n_oth, best_i)


def _sample_body(tok_s_ref, tok_q_ref, a_ref, out_s_ref, out_q_ref, *,
                 width, chunk):
    q = pl.program_id(0)
    a = a_ref[0]
    x0_s = tok_s_ref[0]
    x0_q = tok_q_ref[0]
    row_u = (jnp.uint32(q * width)
             + jax.lax.broadcasted_iota(jnp.uint32, (1, width), 1))
    nb1_s = row_u * jnp.uint32(_STRUC_VOCAB) + jnp.uint32(_KEY_STRUC[1])
    nb1_q = row_u * jnp.uint32(_SEQ_VOCAB) + jnp.uint32(_KEY_SEQ[1])

    m_oth_s, n_oth_s = _scan_others(
        nb1_s, x0_s, n_vocab=_STRUC_VOCAB, n_pad=520, mask_tok=_STRUC_MASK,
        width=width, key=_KEY_STRUC, chunk=chunk)
    m_oth_q, n_oth_q = _scan_others(
        nb1_q, x0_q, n_vocab=_SEQ_VOCAB, n_pad=40, mask_tok=_SEQ_MASK,
        width=width, key=_KEY_SEQ, chunk=chunk)

    s8 = jax.lax.broadcasted_iota(jnp.int32, (8, width), 0)
    sk0, sk1, sk2 = _key_consts(_KEY_STRUC)
    qk0, qk1, qk2 = _key_consts(_KEY_SEQ)
    bc = lambda v: jnp.broadcast_to(v, (8, width))
    x1p = jnp.where(
        s8 == 0, bc(nb1_s + x0_s.astype(jnp.uint32)),
        jnp.where(s8 == 1, bc(nb1_s + jnp.uint32(_STRUC_MASK)),
                  jnp.where(s8 == 2, bc(nb1_q + x0_q.astype(jnp.uint32)),
                            bc(nb1_q + jnp.uint32(_SEQ_MASK)))))
    in_s = s8 < 2
    k0p = jnp.where(in_s, sk0, qk0)
    k1p = jnp.where(in_s, sk1, qk1)
    k2p = jnp.where(in_s, sk2, qk2)
    mbp = _mbits_core(k0p, k1p, k2p, x1p)
    gin = jnp.where(s8 == 4, bc(m_oth_s),
                    jnp.where(s8 == 5, bc(m_oth_q), mbp))
    g = _gumbel_from_mbits(gin)

    out_s_ref[0] = _pick(a, x0_s, g[0:1], g[1:2], g[4:5], n_oth_s,
                         _STRUC_MASK)
    out_q_ref[0] = _pick(a, x0_q, g[2:3], g[3:4], g[5:6], n_oth_q,
                         _SEQ_MASK)


def _run(structure, sequence, a_rows, width=1024, chunk=8, interpret=False):
    rows = int(np.prod(structure.shape))
    grid = rows // width
    tok_s = structure.reshape(grid, 1, width).astype(jnp.int32)
    tok_q = sequence.reshape(grid, 1, width).astype(jnp.int32)
    a3 = a_rows.reshape(grid, 1, width).astype(jnp.float32)
    body = functools.partial(_sample_body, width=width, chunk=chunk)
    spec = pl.BlockSpec((1, 1, width), lambda q: (q, 0, 0))
    out_s, out_q = pl.pallas_call(
        body,
        grid=(grid,),
        in_specs=[spec, spec, spec],
        out_specs=[spec, spec],
        out_shape=[jax.ShapeDtypeStruct((grid, 1, width), jnp.int32),
                   jax.ShapeDtypeStruct((grid, 1, width), jnp.int32)],
        compiler_params=pltpu.CompilerParams(
            dimension_semantics=("parallel",)),
        interpret=interpret,
    )(tok_s, tok_q, a3)
    return out_s.reshape(rows), out_q.reshape(rows)


_ROWS = 65536
_R_SC = 17408
_R_TC = _ROWS - _R_SC
_SC_WORKERS = 32
_SC_RPW = _R_SC // _SC_WORKERS
_SC_GROUPS = _SC_RPW // 16


def _sc_scan_rows(x0v, nb1, key, pre, lo, nblk, post):
    k0, k1, k2 = _key_consts(key)

    def one(n, carry):
        acc, idx = carry
        nvec = jnp.full((16,), n, jnp.int32)
        mb = _mbits_core(k0, k1, k2, nb1 + nvec.astype(jnp.uint32))
        masked = jnp.where(x0v == nvec, jnp.int32(-1), mb)
        upd = masked > acc
        acc = jnp.where(upd, masked, acc)
        idx = jnp.where(upd, nvec, idx)
        return acc, idx

    def blk(i, carry):
        n0 = lo + i * 4
        for j in range(4):
            carry = one(n0 + j, carry)
        return carry

    carry = (jnp.full((16,), -1, jnp.int32), jnp.zeros((16,), jnp.int32))
    for n in pre:
        carry = one(n, carry)
    carry = jax.lax.fori_loop(0, nblk, blk, carry)
    for n in post:
        carry = one(n, carry)
    return carry


def _sc_scan(xs_sc, xq_sc):
    mesh = plsc.VectorSubcoreMesh(core_axis_name="c", subcore_axis_name="s")
    sk = _key_consts(_KEY_STRUC)
    qk = _key_consts(_KEY_SEQ)

    @functools.partial(
        pl.kernel,
        out_type=[jax.ShapeDtypeStruct((_R_SC,), jnp.int32)] * 8,
        mesh=mesh,
        scratch_types=[pltpu.VMEM((_SC_RPW,), jnp.int32)] * 10,
    )
    def body(xs_hbm, xq_hbm, *out_and_scratch):
        out_hbm = out_and_scratch[:8]
        xs_v, xq_v = out_and_scratch[8:10]
        res_v = out_and_scratch[10:]
        wid = jax.lax.axis_index("s") * 2 + jax.lax.axis_index("c")
        base = wid * _SC_RPW
        pltpu.sync_copy(xs_hbm.at[pl.ds(base, _SC_RPW)], xs_v)
        pltpu.sync_copy(xq_hbm.at[pl.ds(base, _SC_RPW)], xq_v)

        def grp(g, carry):
            off = g * 16
            x0s = xs_v[pl.ds(off, 16)]
            x0q = xq_v[pl.ds(off, 16)]
            row = _R_TC + base + off + jax.lax.iota(jnp.int32, 16)
            row_u = row.astype(jnp.uint32)
            nb1_s = row_u * jnp.uint32(_STRUC_VOCAB) + sk[1]
            nb1_q = row_u * jnp.uint32(_SEQ_VOCAB) + qk[1]
            a_s, c_s = _sc_scan_rows(x0s, nb1_s, _KEY_STRUC,
                                     (0, 1), 3, 128, (515,))
            a_q, c_q = _sc_scan_rows(x0q, nb1_q, _KEY_SEQ,
                                     (), 0, 8, ())
            m_xs = _mbits_core(sk[0], sk[1], sk[2],
                               nb1_s + x0s.astype(jnp.uint32))
            m_ms = _mbits_core(sk[0], sk[1], sk[2],
                               nb1_s + jnp.uint32(_STRUC_MASK))
            m_xq = _mbits_core(qk[0], qk[1], qk[2],
                               nb1_q + x0q.astype(jnp.uint32))
            m_mq = _mbits_core(qk[0], qk[1], qk[2],
                               nb1_q + jnp.uint32(_SEQ_MASK))
            for k, v in enumerate((a_s, c_s, m_xs, m_ms,
                                   a_q, c_q, m_xq, m_mq)):
                res_v[k][pl.ds(off, 16)] = v
            return carry

        jax.lax.fori_loop(0, _SC_GROUPS, grp, 0)
        for k in range(8):
            pltpu.sync_copy(res_v[k], out_hbm[k].at[pl.ds(base, _SC_RPW)])

    return body(xs_sc, xq_sc)


def _finish_body(a_ref, xs_ref, xq_ref, mos_ref, nos_ref, mxs_ref, mms_ref,
                 moq_ref, noq_ref, mxq_ref, mmq_ref, out_s_ref, out_q_ref,
                 *, width):
    s8 = jax.lax.broadcasted_iota(jnp.int32, (8, width), 0)
    bc = lambda v: jnp.broadcast_to(v, (8, width))
    gin = jnp.where(
        s8 == 0, bc(mxs_ref[0]),
        jnp.where(s8 == 1, bc(mms_ref[0]),
                  jnp.where(s8 == 2, bc(mos_ref[0]),
                            jnp.where(s8 == 3, bc(mxq_ref[0]),
                                      jnp.where(s8 == 4, bc(mmq_ref[0]),
                                                bc(moq_ref[0]))))))
    g = _gumbel_from_mbits(gin)
    a = a_ref[0]
    out_s_ref[0] = _pick(a, xs_ref[0], g[0:1], g[1:2], g[2:3], nos_ref[0],
                         _STRUC_MASK)
    out_q_ref[0] = _pick(a, xq_ref[0], g[3:4], g[4:5], g[5:6], noq_ref[0],
                         _SEQ_MASK)


def _finish(a_sc, xs_sc, xq_sc, sc_out, width=1024):
    grid = _R_SC // width
    shape3 = (grid, 1, width)
    args = [a_sc.astype(jnp.float32), xs_sc.astype(jnp.int32),
            xq_sc.astype(jnp.int32),
            *sc_out]
    args = [x.reshape(shape3) for x in args]
    spec = pl.BlockSpec((1, 1, width), lambda q: (q, 0, 0))
    out_s, out_q = pl.pallas_call(
        functools.partial(_finish_body, width=width),
        grid=(grid,),
        in_specs=[spec] * 11,
        out_specs=[spec, spec],
        out_shape=[jax.ShapeDtypeStruct(shape3, jnp.int32),
                   jax.ShapeDtypeStruct(shape3, jnp.int32)],
        compiler_params=pltpu.CompilerParams(
            dimension_semantics=("parallel",)),
    )(*args)
    return out_s.reshape(_R_SC), out_q.reshape(_R_SC)


def kernel(structure, sequence, t, alpha):
    b, l = structure.shape
    a = alpha[t]
    a_flat = jnp.broadcast_to(a[:, None], (b, l)).reshape(-1)
    xs = structure.reshape(-1)
    xq = sequence.reshape(-1)
    sc_out = _sc_scan(xs[_R_TC:], xq[_R_TC:])
    tc_s, tc_q = _run(xs[:_R_TC], xq[:_R_TC], a_flat[:_R_TC])
    fin_s, fin_q = _finish(a_flat[_R_TC:], xs[_R_TC:], xq[_R_TC:], sc_out)
    noised_structure = jnp.concatenate([tc_s, fin_s]).reshape(b, l)
    noised_seq = jnp.concatenate([tc_q, fin_q]).reshape(b, l)
    return (noised_structure, noised_seq, t)


# speedup vs baseline: 1.0032x; 1.0032x over previous
import functools

import numpy as np
import jax
import jax.numpy as jnp
from jax.experimental import pallas as pl
from jax.experimental.pallas import tpu as pltpu
from jax.experimental.pallas import tpu_sc as plsc

_STRUC_VOCAB = 516
_SEQ_VOCAB = 33
_STRUC_MASK = 2
_SEQ_MASK = 32
_EPS = 1e-6
_TINY = float(np.finfo(np.float32).tiny)
_BIG = 1 << 30

_KEY_STRUC = (1832780943, 270669613)
_KEY_SEQ = (64467757, 2916123636)

_ROT_A = (13, 15, 26, 6)
_ROT_B = (17, 29, 16, 24)


def _rotl(x, d):
    return (x << jnp.uint32(d)) | (x >> jnp.uint32(32 - d))


def _mbits_core(k0, k1, k2, x1):
    schedule = ((_ROT_A, k1, k2, 1), (_ROT_B, k2, k0, 2),
                (_ROT_A, k0, k1, 3), (_ROT_B, k1, k2, 4),
                (_ROT_A, k2, k0, 5))
    x0 = k0 + x1
    x1 = x0 ^ _rotl(x1, _ROT_A[0])
    first = True
    for rots, ka, kb, c in schedule:
        for r in rots:
            if first:
                first = False
                continue
            x0 = x0 + x1
            x1 = _rotl(x1, r)
            x1 = x0 ^ x1
        x0 = x0 + ka
        x1 = x1 + kb + jnp.uint32(c)
    return ((x0 ^ x1) >> jnp.uint32(9)).astype(jnp.int32)


def _key_consts(key):
    k0, k1 = key
    return jnp.uint32(k0), jnp.uint32(k1), jnp.uint32(k0 ^ k1 ^ 0x1BD11BDA)


def _gumbel_from_mbits(mbits):
    tiny = jnp.float32(_TINY)
    scale = jnp.float32(1.0) - tiny
    fb = jax.lax.bitcast_convert_type(
        mbits | jnp.int32(0x3F800000), jnp.float32) - jnp.float32(1.0)
    u = jnp.maximum(tiny, fb * scale + tiny)
    return -jnp.log(-jnp.log(u))


def _scan_others(nb1, x0, *, n_vocab, n_pad, mask_tok, width, key, chunk):
    k0, k1, k2 = _key_consts(key)
    s_idx = jax.lax.broadcasted_iota(jnp.int32, (chunk, width), 0)
    s_u = jax.lax.broadcasted_iota(jnp.uint32, (chunk, width), 0)
    acc = jnp.full((chunk, width), -1, jnp.int32)
    first_c = jnp.zeros((chunk, width), jnp.int32)
    for c in range(n_pad // chunk):
        n0 = c * chunk
        x1 = nb1 + (s_u + jnp.uint32(n0))
        mb = _mbits_core(k0, k1, k2, x1)
        n_ids = s_idx + n0
        excl = n_ids == x0
        if n0 <= mask_tok < n0 + chunk:
            excl |= n_ids == mask_tok
        if n0 + chunk > n_vocab:
            excl |= n_ids >= n_vocab
        masked = jnp.where(excl, jnp.int32(-1), mb)
        upd = masked > acc
        acc = jnp.where(upd, masked, acc)
        first_c = jnp.where(upd, jnp.int32(c), first_c)
    best_m = jnp.max(acc, axis=0, keepdims=True)
    n_cand = first_c * chunk + s_idx
    elig = acc == best_m
    n_oth = jnp.min(jnp.where(elig, n_cand, jnp.int32(_BIG)),
                    axis=0, keepdims=True)
    return best_m, n_oth


def _pick(a, x0, g_x0, g_mask, g_oth, n_oth, mask_tok):
    eps = jnp.float32(_EPS)
    one = jnp.float32(1.0)
    x0_is_mask = x0 == mask_tok
    p_mask = jnp.where(x0_is_mask, a + (one - a), one - a)
    c_x0 = jnp.where(x0_is_mask, jnp.float32(-jnp.inf),
                     g_x0 + jnp.log(a + eps))
    c_mask = g_mask + jnp.log(p_mask + eps)
    c_oth = g_oth + jnp.log(a * jnp.float32(0.0) + eps)

    best_v, best_i = c_x0, x0
    mask_i = jnp.full_like(best_i, mask_tok)
    take = (c_mask > best_v) | ((c_mask == best_v) & (mask_i < best_i))
    best_v = jnp.where(take, c_mask, best_v)
    best_i = jnp.where(take, mask_i, best_i)
    take = (c_oth > best_v) | ((c_oth == best_v) & (n_oth < best_i))
    return jnp.where(take, n_oth, best_i)


def _sample_body(tok_s_ref, tok_q_ref, a_ref, out_s_ref, out_q_ref, *,
                 width, chunk):
    q = pl.program_id(0)
    a = a_ref[0]
    x0_s = tok_s_ref[0]
    x0_q = tok_q_ref[0]
    row_u = (jnp.uint32(q * width)
             + jax.lax.broadcasted_iota(jnp.uint32, (1, width), 1))
    nb1_s = row_u * jnp.uint32(_STRUC_VOCAB) + jnp.uint32(_KEY_STRUC[1])
    nb1_q = row_u * jnp.uint32(_SEQ_VOCAB) + jnp.uint32(_KEY_SEQ[1])

    m_oth_s, n_oth_s = _scan_others(
        nb1_s, x0_s, n_vocab=_STRUC_VOCAB, n_pad=520, mask_tok=_STRUC_MASK,
        width=width, key=_KEY_STRUC, chunk=chunk)
    m_oth_q, n_oth_q = _scan_others(
        nb1_q, x0_q, n_vocab=_SEQ_VOCAB, n_pad=40, mask_tok=_SEQ_MASK,
        width=width, key=_KEY_SEQ, chunk=chunk)

    s8 = jax.lax.broadcasted_iota(jnp.int32, (8, width), 0)
    sk0, sk1, sk2 = _key_consts(_KEY_STRUC)
    qk0, qk1, qk2 = _key_consts(_KEY_SEQ)
    bc = lambda v: jnp.broadcast_to(v, (8, width))
    x1p = jnp.where(
        s8 == 0, bc(nb1_s + x0_s.astype(jnp.uint32)),
        jnp.where(s8 == 1, bc(nb1_s + jnp.uint32(_STRUC_MASK)),
                  jnp.where(s8 == 2, bc(nb1_q + x0_q.astype(jnp.uint32)),
                            bc(nb1_q + jnp.uint32(_SEQ_MASK)))))
    in_s = s8 < 2
    k0p = jnp.where(in_s, sk0, qk0)
    k1p = jnp.where(in_s, sk1, qk1)
    k2p = jnp.where(in_s, sk2, qk2)
    mbp = _mbits_core(k0p, k1p, k2p, x1p)
    gin = jnp.where(s8 == 4, bc(m_oth_s),
                    jnp.where(s8 == 5, bc(m_oth_q), mbp))
    g = _gumbel_from_mbits(gin)

    out_s_ref[0] = _pick(a, x0_s, g[0:1], g[1:2], g[4:5], n_oth_s,
                         _STRUC_MASK)
    out_q_ref[0] = _pick(a, x0_q, g[2:3], g[3:4], g[5:6], n_oth_q,
                         _SEQ_MASK)


def _run(structure, sequence, a_rows, width=2048, chunk=8, interpret=False):
    rows = int(np.prod(structure.shape))
    grid = rows // width
    tok_s = structure.reshape(grid, 1, width).astype(jnp.int32)
    tok_q = sequence.reshape(grid, 1, width).astype(jnp.int32)
    a3 = a_rows.reshape(grid, 1, width).astype(jnp.float32)
    body = functools.partial(_sample_body, width=width, chunk=chunk)
    spec = pl.BlockSpec((1, 1, width), lambda q: (q, 0, 0))
    out_s, out_q = pl.pallas_call(
        body,
        grid=(grid,),
        in_specs=[spec, spec, spec],
        out_specs=[spec, spec],
        out_shape=[jax.ShapeDtypeStruct((grid, 1, width), jnp.int32),
                   jax.ShapeDtypeStruct((grid, 1, width), jnp.int32)],
        compiler_params=pltpu.CompilerParams(
            dimension_semantics=("parallel",)),
        interpret=interpret,
    )(tok_s, tok_q, a3)
    return out_s.reshape(rows), out_q.reshape(rows)


_ROWS = 65536
_R_SC = 18432
_R_TC = _ROWS - _R_SC
_SC_WORKERS = 32
_SC_RPW = _R_SC // _SC_WORKERS
_SC_GROUPS = _SC_RPW // 16


def _sc_scan_rows(x0v, nb1, key, pre, lo, nblk, post):
    k0, k1, k2 = _key_consts(key)

    def one(n, carry):
        acc, idx = carry
        nvec = jnp.full((16,), n, jnp.int32)
        mb = _mbits_core(k0, k1, k2, nb1 + nvec.astype(jnp.uint32))
        masked = jnp.where(x0v == nvec, jnp.int32(-1), mb)
        upd = masked > acc
        acc = jnp.where(upd, masked, acc)
        idx = jnp.where(upd, nvec, idx)
        return acc, idx

    def blk(i, carry):
        n0 = lo + i * 4
        for j in range(4):
            carry = one(n0 + j, carry)
        return carry

    carry = (jnp.full((16,), -1, jnp.int32), jnp.zeros((16,), jnp.int32))
    for n in pre:
        carry = one(n, carry)
    carry = jax.lax.fori_loop(0, nblk, blk, carry)
    for n in post:
        carry = one(n, carry)
    return carry


def _sc_scan(xs_sc, xq_sc):
    mesh = plsc.VectorSubcoreMesh(core_axis_name="c", subcore_axis_name="s")
    sk = _key_consts(_KEY_STRUC)
    qk = _key_consts(_KEY_SEQ)

    @functools.partial(
        pl.kernel,
        out_type=[jax.ShapeDtypeStruct((_R_SC,), jnp.int32)] * 8,
        mesh=mesh,
        scratch_types=[pltpu.VMEM((_SC_RPW,), jnp.int32)] * 10,
    )
    def body(xs_hbm, xq_hbm, *out_and_scratch):
        out_hbm = out_and_scratch[:8]
        xs_v, xq_v = out_and_scratch[8:10]
        res_v = out_and_scratch[10:]
        wid = jax.lax.axis_index("s") * 2 + jax.lax.axis_index("c")
        base = wid * _SC_RPW
        pltpu.sync_copy(xs_hbm.at[pl.ds(base, _SC_RPW)], xs_v)
        pltpu.sync_copy(xq_hbm.at[pl.ds(base, _SC_RPW)], xq_v)

        def grp(g, carry):
            off = g * 16
            x0s = xs_v[pl.ds(off, 16)]
            x0q = xq_v[pl.ds(off, 16)]
            row = _R_TC + base + off + jax.lax.iota(jnp.int32, 16)
            row_u = row.astype(jnp.uint32)
            nb1_s = row_u * jnp.uint32(_STRUC_VOCAB) + sk[1]
            nb1_q = row_u * jnp.uint32(_SEQ_VOCAB) + qk[1]
            a_s, c_s = _sc_scan_rows(x0s, nb1_s, _KEY_STRUC,
                                     (0, 1), 3, 128, (515,))
            a_q, c_q = _sc_scan_rows(x0q, nb1_q, _KEY_SEQ,
                                     (), 0, 8, ())
            m_xs = _mbits_core(sk[0], sk[1], sk[2],
                               nb1_s + x0s.astype(jnp.uint32))
            m_ms = _mbits_core(sk[0], sk[1], sk[2],
                               nb1_s + jnp.uint32(_STRUC_MASK))
            m_xq = _mbits_core(qk[0], qk[1], qk[2],
                               nb1_q + x0q.astype(jnp.uint32))
            m_mq = _mbits_core(qk[0], qk[1], qk[2],
                               nb1_q + jnp.uint32(_SEQ_MASK))
            for k, v in enumerate((a_s, c_s, m_xs, m_ms,
                                   a_q, c_q, m_xq, m_mq)):
                res_v[k][pl.ds(off, 16)] = v
            return carry

        jax.lax.fori_loop(0, _SC_GROUPS, grp, 0)
        for k in range(8):
            pltpu.sync_copy(res_v[k], out_hbm[k].at[pl.ds(base, _SC_RPW)])

    return body(xs_sc, xq_sc)


def _finish_body(a_ref, xs_ref, xq_ref, mos_ref, nos_ref, mxs_ref, mms_ref,
                 moq_ref, noq_ref, mxq_ref, mmq_ref, out_s_ref, out_q_ref,
                 *, width):
    s8 = jax.lax.broadcasted_iota(jnp.int32, (8, width), 0)
    bc = lambda v: jnp.broadcast_to(v, (8, width))
    gin = jnp.where(
        s8 == 0, bc(mxs_ref[0]),
        jnp.where(s8 == 1, bc(mms_ref[0]),
                  jnp.where(s8 == 2, bc(mos_ref[0]),
                            jnp.where(s8 == 3, bc(mxq_ref[0]),
                                      jnp.where(s8 == 4, bc(mmq_ref[0]),
                                                bc(moq_ref[0]))))))
    g = _gumbel_from_mbits(gin)
    a = a_ref[0]
    out_s_ref[0] = _pick(a, xs_ref[0], g[0:1], g[1:2], g[2:3], nos_ref[0],
                         _STRUC_MASK)
    out_q_ref[0] = _pick(a, xq_ref[0], g[3:4], g[4:5], g[5:6], noq_ref[0],
                         _SEQ_MASK)


def _finish(a_sc, xs_sc, xq_sc, sc_out, width=1024):
    grid = _R_SC // width
    shape3 = (grid, 1, width)
    args = [a_sc.astype(jnp.float32), xs_sc.astype(jnp.int32),
            xq_sc.astype(jnp.int32),
            *sc_out]
    args = [x.reshape(shape3) for x in args]
    spec = pl.BlockSpec((1, 1, width), lambda q: (q, 0, 0))
    out_s, out_q = pl.pallas_call(
        functools.partial(_finish_body, width=width),
        grid=(grid,),
        in_specs=[spec] * 11,
        out_specs=[spec, spec],
        out_shape=[jax.ShapeDtypeStruct(shape3, jnp.int32),
                   jax.ShapeDtypeStruct(shape3, jnp.int32)],
        compiler_params=pltpu.CompilerParams(
            dimension_semantics=("parallel",)),
    )(*args)
    return out_s.reshape(_R_SC), out_q.reshape(_R_SC)


def kernel(structure, sequence, t, alpha):
    b, l = structure.shape
    a = alpha[t]
    a_flat = jnp.broadcast_to(a[:, None], (b, l)).reshape(-1)
    xs = structure.reshape(-1)
    xq = sequence.reshape(-1)
    sc_out = _sc_scan(xs[_R_TC:], xq[_R_TC:])
    tc_s, tc_q = _run(xs[:_R_TC], xq[:_R_TC], a_flat[:_R_TC])
    fin_s, fin_q = _finish(a_flat[_R_TC:], xs[_R_TC:], xq[_R_TC:], sc_out)
    noised_structure = jnp.concatenate([tc_s, fin_s]).reshape(b, l)
    noised_seq = jnp.concatenate([tc_q, fin_q]).reshape(b, l)
    return (noised_structure, noised_seq, t)
